# baseline (speedup 1.0000x reference)
import functools

import jax
import jax.numpy as jnp
from jax import lax
from jax.experimental import pallas as pl
from jax.experimental.pallas import tpu as pltpu
from jax.experimental.pallas import tpu_sc as plsc

NC, NS, L = 2, 16, 16
NW = NC * NS
B = 16384
D = 64
RPW = B // NW
CH = 128
NCHUNK = RPW // CH
REG = 0.01
EPSILON = 0.5


def _sc_stage(users, pos_items, neg_items, user_emb, item_emb):
    mesh = plsc.VectorSubcoreMesh(core_axis_name="c", subcore_axis_name="s",
                                  num_cores=NC, num_subcores=NS)

    @functools.partial(
        pl.kernel,
        out_type=[
            jax.ShapeDtypeStruct((B, L), jnp.float32),
            jax.ShapeDtypeStruct((B, L), jnp.float32),
            jax.ShapeDtypeStruct((B, L), jnp.float32),
            jax.ShapeDtypeStruct((NW, L), jnp.float32),
        ],
        mesh=mesh,
        compiler_params=pltpu.CompilerParams(use_tc_tiling_on_sc=False),
        scratch_types=[
            pltpu.VMEM((NCHUNK, CH), jnp.int32),
            pltpu.VMEM((NCHUNK, CH), jnp.int32),
            pltpu.VMEM((NCHUNK, CH), jnp.int32),
            pltpu.VMEM((RPW, D), jnp.float32),
            pltpu.VMEM((RPW, D), jnp.float32),
            pltpu.VMEM((RPW, D), jnp.float32),
            pltpu.VMEM((RPW, L), jnp.float32),
            pltpu.VMEM((RPW, L), jnp.float32),
            pltpu.VMEM((RPW, L), jnp.float32),
            pltpu.VMEM((1, L), jnp.float32),
            pltpu.SemaphoreType.DMA,
        ],
    )
    def sc_kernel(users_h, pos_h, neg_h, uemb_h, iemb_h,
                  sp_h, up_h, dp_h, pn_h,
                  uidx, pidx, nidx, urows, prows, nrows,
                  sp_v, up_v, dp_v, pn_v, sem):
        w = lax.axis_index("s") * NC + lax.axis_index("c")
        base = w * RPW

        for c in range(NCHUNK):
            off = base + c * CH
            pltpu.sync_copy(users_h.at[pl.ds(off, CH)], uidx.at[c])
            pltpu.sync_copy(pos_h.at[pl.ds(off, CH)], pidx.at[c])
            pltpu.sync_copy(neg_h.at[pl.ds(off, CH)], nidx.at[c])

        copies = []
        for c in range(NCHUNK):
            dst = pl.ds(c * CH, CH)
            copies.append(pltpu.async_copy(uemb_h.at[uidx.at[c]], urows.at[dst], sem))
            copies.append(pltpu.async_copy(iemb_h.at[pidx.at[c]], prows.at[dst], sem))
            copies.append(pltpu.async_copy(iemb_h.at[nidx.at[c]], nrows.at[dst], sem))
        for cp in copies:
            cp.wait()

        zero = jnp.zeros((L,), jnp.float32)

        def row_body(r, pn_acc):
            sp = zero
            up = zero
            dp = zero
            for k in range(D // L):
                sl = pl.ds(k * L, L)
                u = urows[r, sl]
                p = prows[r, sl]
                n = nrows[r, sl]
                dv = p - n
                sp = sp + u * dv
                up = up + u * u
                dp = dp + dv * dv
                pn_acc = pn_acc + (p * p + n * n)
            sp_v[r, :] = sp
            up_v[r, :] = up
            dp_v[r, :] = dp
            return pn_acc

        pn_acc = lax.fori_loop(0, RPW, row_body, zero)
        pn_v[0, :] = pn_acc

        pltpu.sync_copy(sp_v, sp_h.at[pl.ds(base, RPW)])
        pltpu.sync_copy(up_v, up_h.at[pl.ds(base, RPW)])
        pltpu.sync_copy(dp_v, dp_h.at[pl.ds(base, RPW)])
        pltpu.sync_copy(pn_v, pn_h.at[pl.ds(w, 1)])

    return sc_kernel(users, pos_items, neg_items, user_emb, item_emb)


def _tc_body(sp_ref, up_ref, dp_ref, pn_ref, out_ref):
    SP = sp_ref[...]
    UP = up_ref[...]
    DP = dp_ref[...]
    PN = pn_ref[...]
    grp = lax.broadcasted_iota(jnp.int32, (128, 8), 0) // L
    col = lax.broadcasted_iota(jnp.int32, (128, 8), 1)
    M = (grp == col).astype(jnp.float32)
    S = lax.dot(SP, M, preferred_element_type=jnp.float32)
    U2 = lax.dot(UP, M, preferred_element_type=jnp.float32)
    D2 = lax.dot(DP, M, preferred_element_type=jnp.float32)

    g = (-1.0 / B) / (1.0 + jnp.exp(S))
    gsq = g * g
    norm_u = jnp.sqrt(jnp.sum(gsq * D2))
    norm_i = jnp.sqrt(jnp.sum(gsq * U2))
    a = EPSILON / (norm_u + 1e-8)
    b = EPSILON / (norm_i + 1e-8)
    S_adv = S + 2.0 * b * g * U2 + a * g * D2 + 2.0 * a * b * gsq * S

    def logsig(x):
        return jnp.minimum(x, 0.0) - jnp.log1p(jnp.exp(-jnp.abs(x)))

    bpr = -jnp.sum(logsig(S)) / B
    adv = -jnp.sum(logsig(S_adv)) / B
    reg = REG * (jnp.sum(U2) + jnp.sum(PN))
    out_ref[0, 0] = bpr + adv + reg


def _tc_stage(sp, up, dp, pn):
    return pl.pallas_call(
        _tc_body,
        out_shape=jax.ShapeDtypeStruct((1, 1), jnp.float32),
        out_specs=pl.BlockSpec(memory_space=pltpu.SMEM),
    )(sp, up, dp, pn)


def kernel(users, pos_items, neg_items, user_emb, item_emb):
    sp, up, dp, pn = _sc_stage(users, pos_items, neg_items, user_emb, item_emb)
    out = _tc_stage(sp.reshape(B * L // 128, 128),
                    up.reshape(B * L // 128, 128),
                    dp.reshape(B * L // 128, 128),
                    pn.reshape(NW * L // 128, 128))
    return out[0, 0]


# speedup vs baseline: 1.9792x; 1.9792x over previous
import functools

import jax
import jax.numpy as jnp
from jax import lax
from jax.experimental import pallas as pl
from jax.experimental.pallas import tpu as pltpu
from jax.experimental.pallas import tpu_sc as plsc

NC, NS, L = 2, 16, 16
NW = NC * NS
B = 16384
D = 64
TBL = 1000000
SEG = 512
SPW = 62
NSEG = 1954
TAIL_START = (NSEG - 1) * SEG
TAILW = TBL - TAIL_START
CAP = 4080
RPW = B // NW
REG = 0.01
EPSILON = 0.5

_MESH = dict(core_axis_name="c", subcore_axis_name="s",
             num_cores=NC, num_subcores=NS)


def _iota16():
    return lax.broadcasted_iota(jnp.int32, (L,), 0)


def _sc_assemble(users, pos_items, neg_items, t_user, t_item,
                 tail_user, tail_item):
    mesh = plsc.VectorSubcoreMesh(**_MESH)

    @functools.partial(
        pl.kernel,
        out_type=[
            jax.ShapeDtypeStruct((B + 8, 128), jnp.float32),
            jax.ShapeDtypeStruct((B + 8, 128), jnp.float32),
            jax.ShapeDtypeStruct((B + 8, 128), jnp.float32),
        ],
        mesh=mesh,
        compiler_params=pltpu.CompilerParams(needs_layout_passes=False),
        scratch_types=[
            pltpu.VMEM((2, D, SEG), jnp.float32),
            pltpu.VMEM((CAP + L,), jnp.int32),
            pltpu.VMEM((CAP + L,), jnp.int32),
            pltpu.VMEM((CAP + L,), jnp.int32),
            pltpu.VMEM((CAP + L,), jnp.int32),
            pltpu.VMEM((128, 128), jnp.float32),
            pltpu.VMEM((128, 128), jnp.float32),
            pltpu.VMEM((1, 128), jnp.int32),
            pltpu.VMEM((1, 128), jnp.int32),
            pltpu.VMEM((1024,), jnp.int32),
            pltpu.VMEM((64,), jnp.int32),
            pltpu.VMEM((64,), jnp.int32),
            pltpu.VMEM((D, TAILW), jnp.float32),
            pltpu.SemaphoreType.DMA,
            pltpu.SemaphoreType.DMA,
            pltpu.SemaphoreType.DMA,
        ],
    )
    def sc_kernel(users_h, pos_h, neg_h, tu_h, ti_h, tlu_h, tli_h,
                  urows_h, prows_h, nrows_h,
                  buf, larA, ldsA, larB, ldsB, st0, st1, ix0, ix1,
                  scanbuf, prow, pdst, tailbuf, sem0, sem1, semsc):
        w = lax.axis_index("s") * NC + lax.axis_index("c")
        seg_lo = w * SPW
        ii = _iota16()
        zz = jnp.zeros((L,), jnp.int32)

        def seg_id(k):
            return seg_lo + k

        def stream_start(tbl_h, k, parity, sem):
            s = seg_id(k)

            @pl.when((k < SPW) & (s < NSEG - 1))
            def _():
                off = pl.multiple_of(s * SEG, 128)
                pltpu.async_copy(tbl_h.at[:, pl.ds(off, SEG)],
                                 buf.at[parity], sem)

        def stream_wait(tbl_h, k, parity, sem):
            s = seg_id(k)

            @pl.when(s < NSEG - 1)
            def _():
                pltpu.make_async_copy(tbl_h.at[:, pl.ds(0, SEG)],
                                      buf.at[parity], sem).wait()

        def fill(arr_h, lrows, ldest):
            def cbody(c, nl):
                pltpu.sync_copy(arr_h.at[pl.ds(c * 1024, 1024)], scanbuf)

                def vbody(v, nl):
                    rows = scanbuf[pl.ds(v * L, L)]
                    seg = lax.shift_right_logical(rows, 9)
                    m = (seg >= seg_lo) & (seg < seg_lo + SPW)
                    cnt = jnp.sum(m.astype(jnp.int32))
                    ok = nl <= CAP - L
                    pos = nl + plsc.cumsum(m.astype(jnp.int32)) - 1

                    @pl.when(ok & (cnt > 0))
                    def _():
                        dst = c * 1024 + v * L + ii
                        plsc.store_scatter(lrows, [pos], rows, mask=m)
                        plsc.store_scatter(ldest, [pos], dst, mask=m)

                    return lax.select(ok, nl + cnt, nl)

                return lax.fori_loop(0, 1024 // L, vbody, nl)

            return lax.fori_loop(0, B // 1024, cbody, jnp.int32(0))

        def init_ix(ix):
            for g in range(128 // L):
                ix[0, pl.ds(g * L, L)] = jnp.full((L,), B, jnp.int32)

        def emit_chunk(base, m, seg_off, parity, st, ix, out_h, tail):
            r16 = prow[pl.ds(0, L)]
            d16 = pdst[pl.ds(0, L)]
            rl = r16 - seg_off
            pos = base + plsc.cumsum(m.astype(jnp.int32)) - 1
            plsc.store_scatter(ix, [zz, pos], d16, mask=m)
            par = jnp.full((L,), parity, jnp.int32)

            def jbody(jj, _):
                for sub in range(4):
                    js = jnp.full((L,), jj * 4 + sub, jnp.int32)
                    if tail:
                        val = plsc.load_gather(tailbuf, [js, rl], mask=m)
                    else:
                        val = plsc.load_gather(buf, [par, js, rl], mask=m)
                    plsc.store_scatter(st, [pos, js], val, mask=m)
                return 0

            lax.fori_loop(0, D // 4, jbody, 0)

        def flush(st, ix, out_h):
            pltpu.async_copy(st, out_h.at[ix.at[0]], semsc).wait()

        def process_list(lrows, ldest, nlist, s, parity, st, ix, out_h, pend,
                         tail=False):
            seg_off = lax.min(s * SEG, jnp.int32(TAIL_START))
            nv = lax.shift_right_logical(nlist + L - 1, 4)

            def vbody(v, carry):
                npend, pend = carry
                off = v * L
                rows = lrows[pl.ds(off, L)]
                dest = ldest[pl.ds(off, L)]
                valid = ii < (nlist - off)
                if tail:
                    m = valid & (rows >= TAIL_START)
                else:
                    m = (valid & (lax.shift_right_logical(rows, 9) == s)
                         & (rows < TAIL_START))
                cnt = jnp.sum(m.astype(jnp.int32))

                ppos = npend + plsc.cumsum(m.astype(jnp.int32)) - 1

                @pl.when(cnt > 0)
                def _():
                    plsc.store_scatter(prow, [ppos], rows, mask=m)
                    plsc.store_scatter(pdst, [ppos], dest, mask=m)

                np2 = npend + cnt
                full = np2 >= L
                flush_now = full & (pend > 112)
                base = lax.select(flush_now, jnp.int32(0), pend)

                @pl.when(flush_now)
                def _():
                    flush(st, ix, out_h)

                @pl.when(full)
                def _():
                    emit_chunk(base, ii >= 0, seg_off, parity, st, ix, out_h,
                               tail)
                    l1 = prow[pl.ds(L, L)]
                    prow[pl.ds(0, L)] = l1
                    l2 = pdst[pl.ds(L, L)]
                    pdst[pl.ds(0, L)] = l2

                return (lax.select(full, np2 - L, np2),
                        lax.select(full, base + L, pend))

            npend, pend = lax.fori_loop(0, nv, vbody, (jnp.int32(0), pend))
            flush_now = (npend > 0) & (pend > 112)
            base = lax.select(flush_now, jnp.int32(0), pend)

            @pl.when(flush_now)
            def _():
                flush(st, ix, out_h)

            @pl.when(npend > 0)
            def _():
                emit_chunk(base, ii < npend, seg_off, parity, st, ix, out_h,
                           tail)

            return lax.select(npend > 0, base + npend, pend)

        def phase(tbl_h, tail_h, lists):
            stream_start(tbl_h, 0, 0, sem0)
            stream_start(tbl_h, 1, 1, sem1)

            def half(k, parity, sem, pends):
                stream_wait(tbl_h, k, parity, sem)
                s = seg_id(k)
                new_pends = tuple(
                    process_list(lrows, ldest, nlist, s, parity,
                                 st, ix, out_h, pend)
                    for (lrows, ldest, nlist, st, ix, out_h), pend
                    in zip(lists, pends))
                stream_start(tbl_h, k + 2, parity, sem)
                return new_pends

            def pairbody(k2, pends):
                k = 2 * k2
                pends = half(k, 0, sem0, pends)
                pends = half(k + 1, 1, sem1, pends)
                return pends

            pends = lax.fori_loop(0, SPW // 2, pairbody,
                                  tuple(jnp.int32(0) for _ in lists))
            pltpu.sync_copy(tail_h, tailbuf)
            for (lrows, ldest, nlist, st, ix, out_h), pend in zip(lists, pends):
                process_list(lrows, ldest, nlist, jnp.int32(NSEG - 1), 0,
                             st, ix, out_h, pend, tail=True)
                flush(st, ix, out_h)

        init_ix(ix0)
        init_ix(ix1)
        nl_u = fill(users_h, larA, ldsA)
        phase(tu_h, tlu_h, [(larA, ldsA, nl_u, st0, ix0, urows_h)])
        init_ix(ix0)
        init_ix(ix1)
        nl_p = fill(pos_h, larA, ldsA)
        nl_n = fill(neg_h, larB, ldsB)
        phase(ti_h, tli_h, [(larA, ldsA, nl_p, st0, ix0, prows_h),
                            (larB, ldsB, nl_n, st1, ix1, nrows_h)])

    return sc_kernel(users, pos_items, neg_items, t_user, t_item,
                     tail_user, tail_item)


def _sc_reduce(urows, prows, nrows):
    mesh = plsc.VectorSubcoreMesh(**_MESH)

    @functools.partial(
        pl.kernel,
        out_type=[
            jax.ShapeDtypeStruct((B, L), jnp.float32),
            jax.ShapeDtypeStruct((B, L), jnp.float32),
            jax.ShapeDtypeStruct((B, L), jnp.float32),
            jax.ShapeDtypeStruct((NW, L), jnp.float32),
        ],
        mesh=mesh,
        compiler_params=pltpu.CompilerParams(use_tc_tiling_on_sc=False),
        scratch_types=[
            pltpu.VMEM((RPW // 2, 128), jnp.float32),
            pltpu.VMEM((RPW // 2, 128), jnp.float32),
            pltpu.VMEM((RPW // 2, 128), jnp.float32),
            pltpu.VMEM((RPW, L), jnp.float32),
            pltpu.VMEM((RPW, L), jnp.float32),
            pltpu.VMEM((RPW, L), jnp.float32),
            pltpu.VMEM((1, L), jnp.float32),
            pltpu.SemaphoreType.DMA,
        ],
    )
    def sc_kernel(ur_h, pr_h, nr_h, sp_h, up_h, dp_h, pn_h,
                  uv, pv, nv, sp_v, up_v, dp_v, pn_v, sem):
        w = lax.axis_index("s") * NC + lax.axis_index("c")
        base = w * RPW
        half = RPW // 2
        zero = jnp.zeros((L,), jnp.float32)
        pn_acc = zero

        for c in range(2):
            cps = [pltpu.async_copy(ur_h.at[pl.ds(base + c * half, half)],
                                    uv, sem),
                   pltpu.async_copy(pr_h.at[pl.ds(base + c * half, half)],
                                    pv, sem),
                   pltpu.async_copy(nr_h.at[pl.ds(base + c * half, half)],
                                    nv, sem)]
            for cp in cps:
                cp.wait()

            def row_body(r, pn_acc, c=c):
                sp = zero
                up = zero
                dp = zero
                for k in range(D // L):
                    sl = pl.ds(k * L, L)
                    u = uv[r, sl]
                    p = pv[r, sl]
                    n = nv[r, sl]
                    dv = p - n
                    sp = sp + u * dv
                    up = up + u * u
                    dp = dp + dv * dv
                    pn_acc = pn_acc + (p * p + n * n)
                sp_v[c * half + r, :] = sp
                up_v[c * half + r, :] = up
                dp_v[c * half + r, :] = dp
                return pn_acc

            pn_acc = lax.fori_loop(0, half, row_body, pn_acc)

        pn_v[0, :] = pn_acc
        pltpu.sync_copy(sp_v, sp_h.at[pl.ds(base, RPW)])
        pltpu.sync_copy(up_v, up_h.at[pl.ds(base, RPW)])
        pltpu.sync_copy(dp_v, dp_h.at[pl.ds(base, RPW)])
        pltpu.sync_copy(pn_v, pn_h.at[pl.ds(w, 1)])

    return sc_kernel(urows, prows, nrows)


def _tc_body(sp_ref, up_ref, dp_ref, pn_ref, out_ref):
    SP = sp_ref[...]
    UP = up_ref[...]
    DP = dp_ref[...]
    PN = pn_ref[...]
    grp = lax.broadcasted_iota(jnp.int32, (128, 8), 0) // L
    col = lax.broadcasted_iota(jnp.int32, (128, 8), 1)
    M = (grp == col).astype(jnp.float32)
    S = lax.dot(SP, M, preferred_element_type=jnp.float32)
    U2 = lax.dot(UP, M, preferred_element_type=jnp.float32)
    D2 = lax.dot(DP, M, preferred_element_type=jnp.float32)

    g = (-1.0 / B) / (1.0 + jnp.exp(S))
    gsq = g * g
    norm_u = jnp.sqrt(jnp.sum(gsq * D2))
    norm_i = jnp.sqrt(jnp.sum(gsq * U2))
    a = EPSILON / (norm_u + 1e-8)
    b = EPSILON / (norm_i + 1e-8)
    S_adv = S + 2.0 * b * g * U2 + a * g * D2 + 2.0 * a * b * gsq * S

    def logsig(x):
        return jnp.minimum(x, 0.0) - jnp.log1p(jnp.exp(-jnp.abs(x)))

    bpr = -jnp.sum(logsig(S)) / B
    adv = -jnp.sum(logsig(S_adv)) / B
    reg = REG * (jnp.sum(U2) + jnp.sum(PN))
    out_ref[0, 0] = bpr + adv + reg


def _tc_stage(sp, up, dp, pn):
    return pl.pallas_call(
        _tc_body,
        out_shape=jax.ShapeDtypeStruct((1, 1), jnp.float32),
        out_specs=pl.BlockSpec(memory_space=pltpu.SMEM),
    )(sp, up, dp, pn)


def kernel(users, pos_items, neg_items, user_emb, item_emb):
    t_u = jnp.swapaxes(user_emb, 0, 1)
    t_i = jnp.swapaxes(item_emb, 0, 1)
    tail_u = lax.slice(t_u, (0, TAIL_START), (D, TBL))
    tail_i = lax.slice(t_i, (0, TAIL_START), (D, TBL))
    ur, pr, nr = _sc_assemble(users, pos_items, neg_items, t_u, t_i,
                              tail_u, tail_i)
    sp, up, dp, pn = _sc_reduce(ur, pr, nr)
    out = _tc_stage(sp.reshape(B * L // 128, 128),
                    up.reshape(B * L // 128, 128),
                    dp.reshape(B * L // 128, 128),
                    pn.reshape(NW * L // 128, 128))
    return out[0, 0]


# speedup vs baseline: 2.0302x; 1.0258x over previous
import functools

import jax
import jax.numpy as jnp
from jax import lax
from jax.experimental import pallas as pl
from jax.experimental.pallas import tpu as pltpu
from jax.experimental.pallas import tpu_sc as plsc

NC, NS, L = 2, 16, 16
NW = NC * NS
B = 16384
D = 64
TBL = 1000000
SEG = 512
SPW = 62
NSEG = 1954
TAIL_START = (NSEG - 1) * SEG
TAILW = TBL - TAIL_START
CAP = 4080
RPW = B // NW
REG = 0.01
EPSILON = 0.5

_MESH = dict(core_axis_name="c", subcore_axis_name="s",
             num_cores=NC, num_subcores=NS)


def _iota16():
    return lax.broadcasted_iota(jnp.int32, (L,), 0)


def _sc_assemble(users, pos_items, neg_items, t_user, t_item,
                 tail_user, tail_item):
    mesh = plsc.VectorSubcoreMesh(**_MESH)

    @functools.partial(
        pl.kernel,
        out_type=[
            jax.ShapeDtypeStruct((B + 8, 128), jnp.float32),
            jax.ShapeDtypeStruct((B + 8, 128), jnp.float32),
            jax.ShapeDtypeStruct((B + 8, 128), jnp.float32),
        ],
        mesh=mesh,
        compiler_params=pltpu.CompilerParams(needs_layout_passes=False),
        scratch_types=[
            pltpu.VMEM((2, D, SEG), jnp.float32),
            pltpu.VMEM((CAP + L,), jnp.int32),
            pltpu.VMEM((CAP + L,), jnp.int32),
            pltpu.VMEM((CAP + L,), jnp.int32),
            pltpu.VMEM((CAP + L,), jnp.int32),
            pltpu.VMEM((128, 128), jnp.float32),
            pltpu.VMEM((128, 128), jnp.float32),
            pltpu.VMEM((1, 128), jnp.int32),
            pltpu.VMEM((1, 128), jnp.int32),
            pltpu.VMEM((1024,), jnp.int32),
            pltpu.VMEM((64,), jnp.int32),
            pltpu.VMEM((64,), jnp.int32),
            pltpu.VMEM((D, TAILW), jnp.float32),
            pltpu.SemaphoreType.DMA,
            pltpu.SemaphoreType.DMA,
            pltpu.SemaphoreType.DMA,
        ],
    )
    def sc_kernel(users_h, pos_h, neg_h, tu_h, ti_h, tlu_h, tli_h,
                  urows_h, prows_h, nrows_h,
                  buf, larA, ldsA, larB, ldsB, st0, st1, ix0, ix1,
                  scanbuf, prow, pdst, tailbuf, sem0, sem1, semsc):
        w = lax.axis_index("s") * NC + lax.axis_index("c")
        seg_lo = w * SPW
        ii = _iota16()
        zz = jnp.zeros((L,), jnp.int32)

        def seg_id(k):
            return seg_lo + k

        def stream_start(tbl_h, k, parity, sem):
            s = seg_id(k)

            @pl.when((k < SPW) & (s < NSEG - 1))
            def _():
                off = pl.multiple_of(s * SEG, 128)
                pltpu.async_copy(tbl_h.at[:, pl.ds(off, SEG)],
                                 buf.at[parity], sem)

        def stream_wait(tbl_h, k, parity, sem):
            s = seg_id(k)

            @pl.when(s < NSEG - 1)
            def _():
                pltpu.make_async_copy(tbl_h.at[:, pl.ds(0, SEG)],
                                      buf.at[parity], sem).wait()

        def fill(arr_h, lrows, ldest):
            def cbody(c, nl):
                pltpu.sync_copy(arr_h.at[pl.ds(c * 1024, 1024)], scanbuf)

                def vbody(v, nl):
                    rows = scanbuf[pl.ds(v * L, L)]
                    seg = lax.shift_right_logical(rows, 9)
                    m = (seg >= seg_lo) & (seg < seg_lo + SPW)
                    cnt = plsc.all_reduce_population_count(m)[0]
                    ok = nl <= CAP - L

                    @pl.when(ok & (cnt > 0))
                    def _():
                        dst = c * 1024 + v * L + ii
                        pos = nl + plsc.cumsum(m.astype(jnp.int32)) - 1
                        plsc.store_scatter(lrows, [pos], rows, mask=m)
                        plsc.store_scatter(ldest, [pos], dst, mask=m)

                    return lax.select(ok, nl + cnt, nl)

                return lax.fori_loop(0, 1024 // L, vbody, nl)

            return lax.fori_loop(0, B // 1024, cbody, jnp.int32(0))

        def init_ix(ix):
            for g in range(128 // L):
                ix[0, pl.ds(g * L, L)] = jnp.full((L,), B, jnp.int32)

        def emit_chunk(base, m, seg_off, parity, st, ix, out_h, tail):
            r16 = prow[pl.ds(0, L)]
            d16 = pdst[pl.ds(0, L)]
            rl = r16 - seg_off
            pos = base + plsc.cumsum(m.astype(jnp.int32)) - 1
            plsc.store_scatter(ix, [zz, pos], d16, mask=m)
            par = jnp.full((L,), parity, jnp.int32)

            def jbody(jj, _):
                for sub in range(4):
                    js = jnp.full((L,), jj * 4 + sub, jnp.int32)
                    if tail:
                        val = plsc.load_gather(tailbuf, [js, rl], mask=m)
                    else:
                        val = plsc.load_gather(buf, [par, js, rl], mask=m)
                    plsc.store_scatter(st, [pos, js], val, mask=m)
                return 0

            lax.fori_loop(0, D // 4, jbody, 0)

        def flush(st, ix, out_h):
            pltpu.async_copy(st, out_h.at[ix.at[0]], semsc).wait()

        def process_list(lrows, ldest, nlist, s, parity, st, ix, out_h, pend,
                         tail=False):
            seg_off = lax.min(s * SEG, jnp.int32(TAIL_START))
            nv = lax.shift_right_logical(nlist + L - 1, 4)

            def vbody(v, carry):
                npend, pend = carry
                off = v * L
                rows = lrows[pl.ds(off, L)]
                valid = ii < (nlist - off)
                if tail:
                    m = valid & (rows >= TAIL_START)
                else:
                    m = (valid & (lax.shift_right_logical(rows, 9) == s)
                         & (rows < TAIL_START))
                cnt = plsc.all_reduce_population_count(m)[0]
                np2 = npend + cnt
                full = np2 >= L
                flush_now = full & (pend > 112)
                base = lax.select(flush_now, jnp.int32(0), pend)

                @pl.when(cnt > 0)
                def _():
                    dest = ldest[pl.ds(off, L)]
                    ppos = npend + plsc.cumsum(m.astype(jnp.int32)) - 1
                    plsc.store_scatter(prow, [ppos], rows, mask=m)
                    plsc.store_scatter(pdst, [ppos], dest, mask=m)

                    @pl.when(flush_now)
                    def _():
                        flush(st, ix, out_h)

                    @pl.when(full)
                    def _():
                        emit_chunk(base, ii >= 0, seg_off, parity, st, ix,
                                   out_h, tail)
                        l1 = prow[pl.ds(L, L)]
                        prow[pl.ds(0, L)] = l1
                        l2 = pdst[pl.ds(L, L)]
                        pdst[pl.ds(0, L)] = l2

                return (lax.select(full, np2 - L, np2),
                        lax.select(full, base + L, pend))

            npend, pend = lax.fori_loop(0, nv, vbody, (jnp.int32(0), pend))
            flush_now = (npend > 0) & (pend > 112)
            base = lax.select(flush_now, jnp.int32(0), pend)

            @pl.when(flush_now)
            def _():
                flush(st, ix, out_h)

            @pl.when(npend > 0)
            def _():
                emit_chunk(base, ii < npend, seg_off, parity, st, ix, out_h,
                           tail)

            return lax.select(npend > 0, base + npend, pend)

        def phase(tbl_h, tail_h, lists):
            stream_start(tbl_h, 0, 0, sem0)
            stream_start(tbl_h, 1, 1, sem1)

            def half(k, parity, sem, pends):
                stream_wait(tbl_h, k, parity, sem)
                s = seg_id(k)
                new_pends = tuple(
                    process_list(lrows, ldest, nlist, s, parity,
                                 st, ix, out_h, pend)
                    for (lrows, ldest, nlist, st, ix, out_h), pend
                    in zip(lists, pends))
                stream_start(tbl_h, k + 2, parity, sem)
                return new_pends

            def pairbody(k2, pends):
                k = 2 * k2
                pends = half(k, 0, sem0, pends)
                pends = half(k + 1, 1, sem1, pends)
                return pends

            pends = lax.fori_loop(0, SPW // 2, pairbody,
                                  tuple(jnp.int32(0) for _ in lists))
            pltpu.sync_copy(tail_h, tailbuf)
            for (lrows, ldest, nlist, st, ix, out_h), pend in zip(lists, pends):
                process_list(lrows, ldest, nlist, jnp.int32(NSEG - 1), 0,
                             st, ix, out_h, pend, tail=True)
                flush(st, ix, out_h)

        init_ix(ix0)
        init_ix(ix1)
        nl_u = fill(users_h, larA, ldsA)
        phase(tu_h, tlu_h, [(larA, ldsA, nl_u, st0, ix0, urows_h)])
        init_ix(ix0)
        init_ix(ix1)
        nl_p = fill(pos_h, larA, ldsA)
        nl_n = fill(neg_h, larB, ldsB)
        phase(ti_h, tli_h, [(larA, ldsA, nl_p, st0, ix0, prows_h),
                            (larB, ldsB, nl_n, st1, ix1, nrows_h)])

    return sc_kernel(users, pos_items, neg_items, t_user, t_item,
                     tail_user, tail_item)


def _sc_reduce(urows, prows, nrows):
    mesh = plsc.VectorSubcoreMesh(**_MESH)

    @functools.partial(
        pl.kernel,
        out_type=[
            jax.ShapeDtypeStruct((B, L), jnp.float32),
            jax.ShapeDtypeStruct((B, L), jnp.float32),
            jax.ShapeDtypeStruct((B, L), jnp.float32),
            jax.ShapeDtypeStruct((NW, L), jnp.float32),
        ],
        mesh=mesh,
        compiler_params=pltpu.CompilerParams(use_tc_tiling_on_sc=False),
        scratch_types=[
            pltpu.VMEM((RPW // 2, 128), jnp.float32),
            pltpu.VMEM((RPW // 2, 128), jnp.float32),
            pltpu.VMEM((RPW // 2, 128), jnp.float32),
            pltpu.VMEM((RPW, L), jnp.float32),
            pltpu.VMEM((RPW, L), jnp.float32),
            pltpu.VMEM((RPW, L), jnp.float32),
            pltpu.VMEM((1, L), jnp.float32),
            pltpu.SemaphoreType.DMA,
        ],
    )
    def sc_kernel(ur_h, pr_h, nr_h, sp_h, up_h, dp_h, pn_h,
                  uv, pv, nv, sp_v, up_v, dp_v, pn_v, sem):
        w = lax.axis_index("s") * NC + lax.axis_index("c")
        base = w * RPW
        half = RPW // 2
        zero = jnp.zeros((L,), jnp.float32)
        pn_acc = zero

        for c in range(2):
            cps = [pltpu.async_copy(ur_h.at[pl.ds(base + c * half, half)],
                                    uv, sem),
                   pltpu.async_copy(pr_h.at[pl.ds(base + c * half, half)],
                                    pv, sem),
                   pltpu.async_copy(nr_h.at[pl.ds(base + c * half, half)],
                                    nv, sem)]
            for cp in cps:
                cp.wait()

            def row_body(r, pn_acc, c=c):
                sp = zero
                up = zero
                dp = zero
                for k in range(D // L):
                    sl = pl.ds(k * L, L)
                    u = uv[r, sl]
                    p = pv[r, sl]
                    n = nv[r, sl]
                    dv = p - n
                    sp = sp + u * dv
                    up = up + u * u
                    dp = dp + dv * dv
                    pn_acc = pn_acc + (p * p + n * n)
                sp_v[c * half + r, :] = sp
                up_v[c * half + r, :] = up
                dp_v[c * half + r, :] = dp
                return pn_acc

            pn_acc = lax.fori_loop(0, half, row_body, pn_acc)

        pn_v[0, :] = pn_acc
        pltpu.sync_copy(sp_v, sp_h.at[pl.ds(base, RPW)])
        pltpu.sync_copy(up_v, up_h.at[pl.ds(base, RPW)])
        pltpu.sync_copy(dp_v, dp_h.at[pl.ds(base, RPW)])
        pltpu.sync_copy(pn_v, pn_h.at[pl.ds(w, 1)])

    return sc_kernel(urows, prows, nrows)


def _tc_body(sp_ref, up_ref, dp_ref, pn_ref, out_ref):
    SP = sp_ref[...]
    UP = up_ref[...]
    DP = dp_ref[...]
    PN = pn_ref[...]
    grp = lax.broadcasted_iota(jnp.int32, (128, 8), 0) // L
    col = lax.broadcasted_iota(jnp.int32, (128, 8), 1)
    M = (grp == col).astype(jnp.float32)
    S = lax.dot(SP, M, preferred_element_type=jnp.float32)
    U2 = lax.dot(UP, M, preferred_element_type=jnp.float32)
    D2 = lax.dot(DP, M, preferred_element_type=jnp.float32)

    g = (-1.0 / B) / (1.0 + jnp.exp(S))
    gsq = g * g
    norm_u = jnp.sqrt(jnp.sum(gsq * D2))
    norm_i = jnp.sqrt(jnp.sum(gsq * U2))
    a = EPSILON / (norm_u + 1e-8)
    b = EPSILON / (norm_i + 1e-8)
    S_adv = S + 2.0 * b * g * U2 + a * g * D2 + 2.0 * a * b * gsq * S

    def logsig(x):
        return jnp.minimum(x, 0.0) - jnp.log1p(jnp.exp(-jnp.abs(x)))

    bpr = -jnp.sum(logsig(S)) / B
    adv = -jnp.sum(logsig(S_adv)) / B
    reg = REG * (jnp.sum(U2) + jnp.sum(PN))
    out_ref[0, 0] = bpr + adv + reg


def _tc_stage(sp, up, dp, pn):
    return pl.pallas_call(
        _tc_body,
        out_shape=jax.ShapeDtypeStruct((1, 1), jnp.float32),
        out_specs=pl.BlockSpec(memory_space=pltpu.SMEM),
    )(sp, up, dp, pn)


def kernel(users, pos_items, neg_items, user_emb, item_emb):
    t_u = jnp.swapaxes(user_emb, 0, 1)
    t_i = jnp.swapaxes(item_emb, 0, 1)
    tail_u = lax.slice(t_u, (0, TAIL_START), (D, TBL))
    tail_i = lax.slice(t_i, (0, TAIL_START), (D, TBL))
    ur, pr, nr = _sc_assemble(users, pos_items, neg_items, t_u, t_i,
                              tail_u, tail_i)
    sp, up, dp, pn = _sc_reduce(ur, pr, nr)
    out = _tc_stage(sp.reshape(B * L // 128, 128),
                    up.reshape(B * L // 128, 128),
                    dp.reshape(B * L // 128, 128),
                    pn.reshape(NW * L // 128, 128))
    return out[0, 0]
